# baseline (speedup 1.0000x reference)
import functools

import jax
import jax.numpy as jnp
from jax import lax
from jax.experimental import pallas as pl
from jax.experimental.pallas import tpu as pltpu
from jax.experimental.pallas import tpu_sc as plsc

B = 16384
L = 200
D = 32
VOCAB = 1000000
NC = 2
NS = 16
NW = NC * NS
G = B // NW
ROWS_PER_TILE = G * L
CHUNK = 1024
NCHUNK = ROWS_PER_TILE // CHUNK
TRASH = NS * G


def _body(ids_hbm, len_hbm, tab_hbm, out_hbm,
          acc_sp, idsb_a, idsb_b, rb_a, rb_b, sv_a, sv_b,
          len_v, inv_v, obuf, semg, sems0, sems1):
    c = lax.axis_index("c")
    s = lax.axis_index("s")
    wid = c * NS + s
    base_b = wid * G
    slot0 = s * G
    idsb = (idsb_a, idsb_b)
    rb = (rb_a, rb_b)
    sv = (sv_a, sv_b)
    sems = (sems0, sems1)

    iota = lax.iota(jnp.int32, 16)

    pltpu.sync_copy(len_hbm.at[pl.ds(base_b, G)], len_v)
    for k in range(G // 16):
        lv = len_v[pl.ds(k * 16, 16)]
        inv_v[pl.ds(k * 16, 16)] = 1.0 / lv.astype(jnp.float32)

    zero = jnp.zeros((16,), jnp.float32)
    for r in range(128):
        for h in range(D // 16):
            obuf[r, pl.ds(h * 16, 16)] = zero
    for p in range(G // 128):
        pltpu.sync_copy(obuf, acc_sp.at[pl.ds(slot0 + p * 128, 128)])

    for sub in range(2):
        for off in (200, 216, 232, 240):
            idsb[sub][pl.ds(off, 16)] = iota * 0

    def length_of(b):
        off = pl.multiple_of((b // 16) * 16, 8)
        lens16 = len_v[pl.ds(off, 16)]
        lane = b - off
        return jnp.max(jnp.where(iota == lane, lens16, 0))

    def stage_ids(b, sub):
        pltpu.sync_copy(ids_hbm.at[base_b + b], idsb[sub].at[pl.ds(0, L)])

    def fire_gathers(b, sub):
        lb = length_of(b)
        for j in range(4):
            @pl.when((j == 0) | (j * 64 < lb))
            def _():
                pltpu.async_copy(tab_hbm.at[idsb[sub].at[pl.ds(j * 64, 64)]],
                                 rb[sub].at[pl.ds(j * 64, 64)], semg)

    def drain(b, sub, sem):
        lb = length_of(b)
        for j in range(4):
            @pl.when((j == 0) | (j * 64 < lb))
            def _():
                pltpu.make_async_copy(tab_hbm.at[pl.ds(0, 64)],
                                      rb[sub].at[pl.ds(j * 64, 64)],
                                      sem).wait()

    def compute_seg(b, sub):
        lb = length_of(b)
        slot = jnp.broadcast_to(slot0 + b, (16,))
        trash = jnp.broadcast_to(TRASH + s, (16,))
        lbv = jnp.broadcast_to(lb, (16,))
        for j in range(4):
            for k in range(4):
                l = iota + (j * 64 + k * 16)
                seg = jnp.where(l < lbv, slot, trash)
                sv[sub][j, pl.ds(k * 16, 16)] = seg

    def fire_scatters(b, sub):
        lb = length_of(b)
        for j in range(4):
            @pl.when((j == 0) | (j * 64 < lb))
            def _():
                pltpu.async_copy(rb[sub].at[pl.ds(j * 64, 64)],
                                 acc_sp.at[sv[sub].at[j]], sems[sub],
                                 add=True)

    stage_ids(0, 0)
    fire_gathers(0, 0)

    @pl.loop(0, G, step=2)
    def _row(t):
        for sub in range(2):
            b = t + sub
            compute_seg(b, sub)
            drain(b, sub, semg)
            fire_scatters(b, sub)

            @pl.when(b < G - 1)
            def _prep():
                @pl.when(b >= 1)
                def _free():
                    drain(b - 1, 1 - sub, sems[1 - sub])
                stage_ids(b + 1, 1 - sub)
                fire_gathers(b + 1, 1 - sub)

    drain(G - 2, 0, sems[0])
    drain(G - 1, 1, sems[1])

    for p in range(G // 128):
        pltpu.sync_copy(acc_sp.at[pl.ds(slot0 + p * 128, 128)], obuf)

        @pl.loop(0, 128)
        def _scale(b):
            inv = plsc.load_gather(inv_v, [jnp.broadcast_to(p * 128 + b, (16,))])
            for h in range(D // 16):
                obuf[b, pl.ds(h * 16, 16)] = obuf[b, pl.ds(h * 16, 16)] * inv

        pltpu.sync_copy(obuf, out_hbm.at[pl.ds(base_b + p * 128, 128)])


NFULL = 7812
TAIL = VOCAB - NFULL * 128


def _tr_body(embt_hbm, tail_hbm, zero_hbm, out_hbm,
             st_a, st_b, st2_a, st2_b, tailv, zv, semi, semo):
    c = lax.axis_index("c")
    s = lax.axis_index("s")
    wid = c * NS + s
    base = NFULL // 32
    extra = NFULL - 32 * base
    nb = jnp.where(wid < extra, base + 1, base)
    start = wid * base + jnp.minimum(wid, extra)

    st = (st_a, st_b)
    st2 = (st2_a, st2_b)
    iota32 = lax.iota(jnp.int32, 16) * D

    def load_block(bi, sub):
        for dt in range(D // 8):
            pltpu.async_copy(
                embt_hbm.at[pl.ds(dt * 8, 8), pl.ds(bi * 128, 128)],
                st[sub].at[pl.ds(dt * 8, 8)], semi)

    def drain_in(sub):
        for dt in range(D // 8):
            pltpu.make_async_copy(
                embt_hbm.at[pl.ds(dt * 8, 8), pl.ds(0, 128)],
                st[sub].at[pl.ds(dt * 8, 8)], semi).wait()

    def drain_out(sub):
        pltpu.make_async_copy(out_hbm.at[pl.ds(0, 128 * D)],
                              st2[sub], semo).wait()

    pltpu.sync_copy(zero_hbm, zv)
    iota_z = lax.iota(jnp.int32, 16) + zv[pl.ds(0, 16)]

    def transpose_block(sub):
        for d0 in range(0, D, 16):
            drow = iota_z + d0

            @pl.loop(0, 16, unroll=8)
            def _j(j):
                perm = (iota_z + j) & 15
                st_base = perm * D + drow
                for cg in range(8):
                    c0 = cg * 16
                    x = plsc.load_gather(st[sub], [drow, perm + c0])
                    plsc.store_scatter(st2[sub], [st_base + c0 * D], x)

    def store_block(bi, sub):
        pltpu.async_copy(st2[sub],
                         out_hbm.at[pl.ds(bi * (128 * D), 128 * D)], semo)

    load_block(start, 0)

    @pl.loop(0, base + 1, step=2)
    def _blk(t):
        for sub in range(2):
            g = t + sub

            @pl.when(g < nb)
            def _():
                drain_in(sub)
                @pl.when(g + 1 < nb)
                def _():
                    load_block(start + g + 1, 1 - sub)
                @pl.when(g >= 2)
                def _():
                    drain_out(sub)
                transpose_block(sub)
                store_block(start + g, sub)

    drain_out(0)
    drain_out(1)

    @pl.when(wid == 31)
    def _():
        pltpu.sync_copy(tail_hbm, tailv)
        pltpu.sync_copy(tailv, out_hbm.at[pl.ds(NFULL * 128 * D, TAIL * D)])


_transpose = pl.kernel(
    _tr_body,
    out_type=jax.ShapeDtypeStruct((VOCAB * D,), jnp.float32),
    mesh=plsc.VectorSubcoreMesh(core_axis_name="c", subcore_axis_name="s"),
    compiler_params=pltpu.CompilerParams(
        needs_layout_passes=False, use_tc_tiling_on_sc=True),
    scratch_types=[
        pltpu.VMEM((D, 128), jnp.float32),
        pltpu.VMEM((D, 128), jnp.float32),
        pltpu.VMEM((128 * D,), jnp.float32),
        pltpu.VMEM((128 * D,), jnp.float32),
        pltpu.VMEM((TAIL * D,), jnp.float32),
        pltpu.VMEM((16,), jnp.int32),
        pltpu.SemaphoreType.DMA,
        pltpu.SemaphoreType.DMA,
    ],
)


_bag = pl.kernel(
    _body,
    out_type=jax.ShapeDtypeStruct((B, D), jnp.float32),
    mesh=plsc.VectorSubcoreMesh(core_axis_name="c", subcore_axis_name="s"),
    compiler_params=pltpu.CompilerParams(
        needs_layout_passes=False, use_tc_tiling_on_sc=False),
    scratch_types=[
        pltpu.VMEM_SHARED((NS * G + NS, D), jnp.float32),
        pltpu.VMEM((4 * 64,), jnp.int32),
        pltpu.VMEM((4 * 64,), jnp.int32),
        pltpu.VMEM((4 * 64, D), jnp.float32),
        pltpu.VMEM((4 * 64, D), jnp.float32),
        pltpu.VMEM((4, 64), jnp.int32),
        pltpu.VMEM((4, 64), jnp.int32),
        pltpu.VMEM((G,), jnp.int32),
        pltpu.VMEM((G,), jnp.float32),
        pltpu.VMEM((128, D), jnp.float32),
        pltpu.SemaphoreType.DMA,
        pltpu.SemaphoreType.DMA,
        pltpu.SemaphoreType.DMA,
    ],
)


@jax.jit
def kernel(ngram_ids, ngram_lengths, embedding):
    emb_t = embedding.T
    tail_flat = embedding[NFULL * 128:, :].reshape(TAIL * D)
    emb_lin = _transpose(emb_t, tail_flat, jnp.zeros((16,), jnp.int32))
    return _bag(ngram_ids, ngram_lengths, emb_lin.reshape(VOCAB, D))


# speedup vs baseline: 1.1370x; 1.1370x over previous
import functools

import jax
import jax.numpy as jnp
from jax import lax
from jax.experimental import pallas as pl
from jax.experimental.pallas import tpu as pltpu
from jax.experimental.pallas import tpu_sc as plsc

B = 16384
L = 200
D = 32
VOCAB = 1000000
NC = 2
NS = 16
NW = NC * NS
G = B // NW
ROWS_PER_TILE = G * L
CHUNK = 1024
NCHUNK = ROWS_PER_TILE // CHUNK
TRASH = NS * G


CB = 2
SLOTS = CB * 256
NCHUNK2 = G // CB


def _body(ids_hbm, len_hbm, tab_hbm, out_hbm,
          acc_sp, ids_v, seg_v, rows_v, len_v, inv_v, obuf,
          semg, sems0, sems1):
    c = lax.axis_index("c")
    s = lax.axis_index("s")
    wid = c * NS + s
    base_b = wid * G
    slot0 = s * G
    sems = (sems0, sems1)

    iota = lax.iota(jnp.int32, 16)

    pltpu.sync_copy(len_hbm.at[pl.ds(base_b, G)], len_v)
    for k in range(G // 16):
        lv = len_v[pl.ds(k * 16, 16)]
        inv_v[pl.ds(k * 16, 16)] = 1.0 / lv.astype(jnp.float32)

    zvec = jnp.where(len_v[pl.ds(0, 16)] < 0, len_v[pl.ds(0, 16)], 0)
    iota_z = iota + zvec

    zero = jnp.zeros((16,), jnp.float32)
    for r in range(128):
        for h in range(D // 16):
            obuf[r, pl.ds(h * 16, 16)] = zero
    for p in range(G // 128):
        pltpu.sync_copy(obuf, acc_sp.at[pl.ds(slot0 + p * 128, 128)])

    for sub in range(2):
        for i in range(CB):
            for off in (200, 216, 232, 240):
                ids_v[sub, pl.ds(i * 256 + off, 16)] = iota * 0

    def lens_of(g):
        off = pl.multiple_of((g // 8) * 16, 8)
        lens16 = len_v[pl.ds(off, 16)]
        lane0 = g * CB - off
        lb0 = jnp.max(jnp.where(iota == lane0, lens16, 0))
        lb1 = jnp.max(jnp.where(iota == lane0 + 1, lens16, 0))
        return (lb0, lb1)

    def stage_ids(g, sub):
        for i in range(CB):
            pltpu.sync_copy(ids_hbm.at[base_b + g * CB + i],
                            ids_v.at[sub].at[pl.ds(i * 256, L)])

    def fire_gathers(g, sub):
        lb = lens_of(g)
        for i in range(CB):
            for j in range(4):
                def fire(i=i, j=j):
                    o = i * 256 + j * 64
                    pltpu.async_copy(
                        tab_hbm.at[ids_v.at[sub].at[pl.ds(o, 64)]],
                        rows_v.at[sub].at[pl.ds(o, 64)], semg)
                if j == 0:
                    fire()
                else:
                    pl.when(j * 64 < lb[i])(fire)

    def drain_g(g, sub, sem):
        lb = lens_of(g)
        for i in range(CB):
            for j in range(4):
                def wait(i=i, j=j):
                    o = i * 256 + j * 64
                    pltpu.make_async_copy(
                        tab_hbm.at[pl.ds(0, 64)],
                        rows_v.at[sub].at[pl.ds(o, 64)], sem).wait()
                if j == 0:
                    wait()
                else:
                    pl.when(j * 64 < lb[i])(wait)

    def compute_seg(g, sub):
        lb = lens_of(g)
        trash = jnp.broadcast_to(TRASH + s, (16,))
        for i in range(CB):
            lbv = jnp.broadcast_to(lb[i], (16,))
            slot = jnp.broadcast_to(slot0 + g * CB + i, (16,))
            for k in range(16):
                l = iota_z + k * 16
                seg = jnp.where(l < lbv, slot, trash)
                seg_v[sub, i * 4 + k // 4, pl.ds((k % 4) * 16, 16)] = seg

    def fire_scatters(g, sub):
        lb = lens_of(g)
        for i in range(CB):
            for j in range(4):
                def fire(i=i, j=j):
                    o = i * 256 + j * 64
                    pltpu.async_copy(
                        rows_v.at[sub].at[pl.ds(o, 64)],
                        acc_sp.at[seg_v.at[sub].at[i * 4 + j]], sems[sub],
                        add=True)
                if j == 0:
                    fire()
                else:
                    pl.when(j * 64 < lb[i])(fire)

    stage_ids(0, 0)
    fire_gathers(0, 0)

    @pl.loop(0, NCHUNK2, step=2)
    def _chunk(go):
        for sub in range(2):
            g = go + sub
            compute_seg(g, sub)
            drain_g(g, sub, semg)
            fire_scatters(g, sub)

            @pl.when(g < NCHUNK2 - 1)
            def _prep():
                @pl.when(g >= 1)
                def _free():
                    drain_g(g - 1, 1 - sub, sems[1 - sub])
                stage_ids(g + 1, 1 - sub)
                fire_gathers(g + 1, 1 - sub)

    drain_g(NCHUNK2 - 2, 0, sems[0])
    drain_g(NCHUNK2 - 1, 1, sems[1])

    for p in range(G // 128):
        pltpu.sync_copy(acc_sp.at[pl.ds(slot0 + p * 128, 128)], obuf)

        @pl.loop(0, 128)
        def _scale(b):
            inv = plsc.load_gather(inv_v, [jnp.broadcast_to(p * 128 + b, (16,))])
            for h in range(D // 16):
                obuf[b, pl.ds(h * 16, 16)] = obuf[b, pl.ds(h * 16, 16)] * inv

        pltpu.sync_copy(obuf, out_hbm.at[pl.ds(base_b + p * 128, 128)])


NFULL = 7812
TAIL = VOCAB - NFULL * 128


def _tr_body(embt_hbm, tail_hbm, zero_hbm, out_hbm,
             st_a, st_b, st2_a, st2_b, tailv, zv, semi, semo):
    c = lax.axis_index("c")
    s = lax.axis_index("s")
    wid = c * NS + s
    base = NFULL // 32
    extra = NFULL - 32 * base
    nb = jnp.where(wid < extra, base + 1, base)
    start = wid * base + jnp.minimum(wid, extra)

    st = (st_a, st_b)
    st2 = (st2_a, st2_b)
    iota32 = lax.iota(jnp.int32, 16) * D

    def load_block(bi, sub):
        for dt in range(D // 8):
            pltpu.async_copy(
                embt_hbm.at[pl.ds(dt * 8, 8), pl.ds(bi * 128, 128)],
                st[sub].at[pl.ds(dt * 8, 8)], semi)

    def drain_in(sub):
        for dt in range(D // 8):
            pltpu.make_async_copy(
                embt_hbm.at[pl.ds(dt * 8, 8), pl.ds(0, 128)],
                st[sub].at[pl.ds(dt * 8, 8)], semi).wait()

    def drain_out(sub):
        pltpu.make_async_copy(out_hbm.at[pl.ds(0, 128 * D)],
                              st2[sub], semo).wait()

    pltpu.sync_copy(zero_hbm, zv)
    iota_z = lax.iota(jnp.int32, 16) + zv[pl.ds(0, 16)]

    def transpose_block(sub):
        for d0 in range(0, D, 16):
            drow = iota_z + d0

            @pl.loop(0, 16, unroll=8)
            def _j(j):
                perm = (iota_z + j) & 15
                st_base = perm * D + drow
                for cg in range(8):
                    c0 = cg * 16
                    x = plsc.load_gather(st[sub], [drow, perm + c0])
                    plsc.store_scatter(st2[sub], [st_base + c0 * D], x)

    def store_block(bi, sub):
        pltpu.async_copy(st2[sub],
                         out_hbm.at[pl.ds(bi * (128 * D), 128 * D)], semo)

    load_block(start, 0)

    @pl.loop(0, base + 1, step=2)
    def _blk(t):
        for sub in range(2):
            g = t + sub

            @pl.when(g < nb)
            def _():
                drain_in(sub)
                @pl.when(g + 1 < nb)
                def _():
                    load_block(start + g + 1, 1 - sub)
                @pl.when(g >= 2)
                def _():
                    drain_out(sub)
                transpose_block(sub)
                store_block(start + g, sub)

    drain_out(0)
    drain_out(1)

    @pl.when(wid == 31)
    def _():
        pltpu.sync_copy(tail_hbm, tailv)
        pltpu.sync_copy(tailv, out_hbm.at[pl.ds(NFULL * 128 * D, TAIL * D)])


_transpose = pl.kernel(
    _tr_body,
    out_type=jax.ShapeDtypeStruct((VOCAB * D,), jnp.float32),
    mesh=plsc.VectorSubcoreMesh(core_axis_name="c", subcore_axis_name="s"),
    compiler_params=pltpu.CompilerParams(
        needs_layout_passes=False, use_tc_tiling_on_sc=True),
    scratch_types=[
        pltpu.VMEM((D, 128), jnp.float32),
        pltpu.VMEM((D, 128), jnp.float32),
        pltpu.VMEM((128 * D,), jnp.float32),
        pltpu.VMEM((128 * D,), jnp.float32),
        pltpu.VMEM((TAIL * D,), jnp.float32),
        pltpu.VMEM((16,), jnp.int32),
        pltpu.SemaphoreType.DMA,
        pltpu.SemaphoreType.DMA,
    ],
)


_bag = pl.kernel(
    _body,
    out_type=jax.ShapeDtypeStruct((B, D), jnp.float32),
    mesh=plsc.VectorSubcoreMesh(core_axis_name="c", subcore_axis_name="s"),
    compiler_params=pltpu.CompilerParams(
        needs_layout_passes=False, use_tc_tiling_on_sc=False),
    scratch_types=[
        pltpu.VMEM_SHARED((NS * G + NS, D), jnp.float32),
        pltpu.VMEM((2, SLOTS), jnp.int32),
        pltpu.VMEM((2, SLOTS // 64, 64), jnp.int32),
        pltpu.VMEM((2, SLOTS, D), jnp.float32),
        pltpu.VMEM((G,), jnp.int32),
        pltpu.VMEM((G,), jnp.float32),
        pltpu.VMEM((128, D), jnp.float32),
        pltpu.SemaphoreType.DMA,
        pltpu.SemaphoreType.DMA,
        pltpu.SemaphoreType.DMA,
    ],
)


@jax.jit
def kernel(ngram_ids, ngram_lengths, embedding):
    emb_t = embedding.T
    tail_flat = embedding[NFULL * 128:, :].reshape(TAIL * D)
    emb_lin = _transpose(emb_t, tail_flat, jnp.zeros((16,), jnp.int32))
    return _bag(ngram_ids, ngram_lengths, emb_lin.reshape(VOCAB, D))


# speedup vs baseline: 1.6569x; 1.4572x over previous
import functools

import jax
import jax.numpy as jnp
from jax import lax
from jax.experimental import pallas as pl
from jax.experimental.pallas import tpu as pltpu
from jax.experimental.pallas import tpu_sc as plsc

B = 16384
L = 200
D = 32
VOCAB = 1000000
NC = 2
NS = 16
NW = NC * NS
G = B // NW
ROWS_PER_TILE = G * L
CHUNK = 1024
NCHUNK = ROWS_PER_TILE // CHUNK
TRASH = NS * G


def _body(ids_hbm, len_hbm, tab_hbm, out_hbm,
          acc_sp, ids_v, seg_v, rows_v, len_v, inv_v, obuf,
          semg, sems0, sems1):
    c = lax.axis_index("c")
    s = lax.axis_index("s")
    wid = c * NS + s
    base_b = wid * G
    slot0 = s * G
    sems = (sems0, sems1)
    NDMA = CHUNK // 128

    pltpu.sync_copy(len_hbm.at[pl.ds(base_b, G)], len_v)
    for k in range(G // 16):
        lv = len_v[pl.ds(k * 16, 16)]
        inv_v[pl.ds(k * 16, 16)] = 1.0 / lv.astype(jnp.float32)

    zero = jnp.zeros((16,), jnp.float32)
    for r in range(128):
        for h in range(D // 16):
            obuf[r, pl.ds(h * 16, 16)] = zero
    for p in range(G // 128):
        pltpu.sync_copy(obuf, acc_sp.at[pl.ds(slot0 + p * 128, 128)])

    iota = lax.iota(jnp.int32, 16)

    def stage_ids(g, sub):
        idrow = wid * (ROWS_PER_TILE // 128) + g * NDMA
        pltpu.sync_copy(ids_hbm.at[pl.ds(idrow, NDMA)], ids_v.at[sub])

    def fire_gathers(sub):
        for j in range(NDMA):
            pltpu.async_copy(tab_hbm.at[ids_v.at[sub].at[j]],
                             rows_v.at[sub].at[pl.ds(j * 128, 128)], semg)

    def drain(sem, sub):
        pltpu.make_async_copy(tab_hbm.at[pl.ds(0, CHUNK)],
                              rows_v.at[sub], sem).wait()

    def compute_seg(g, sub):
        row0 = wid * ROWS_PER_TILE + g * CHUNK
        for j in range(NDMA):
            for k in range(8):
                gidx = row0 + (j * 8 + k) * 16 + iota
                q = lax.div(gidx, L)
                l = gidx - q * L
                bl = q - base_b
                lens = plsc.load_gather(len_v, [bl])
                seg = jnp.where(l < lens, bl + slot0, TRASH + s)
                seg_v[sub, j, pl.ds(k * 16, 16)] = seg

    def fire_scatters(sub):
        for j in range(NDMA):
            pltpu.async_copy(rows_v.at[sub].at[pl.ds(j * 128, 128)],
                             acc_sp.at[seg_v.at[sub].at[j]], sems[sub],
                             add=True)

    stage_ids(0, 0)
    fire_gathers(0)

    @pl.loop(0, NCHUNK, step=2)
    def _chunk(go):
        for sub in range(2):
            g = go + sub
            compute_seg(g, sub)
            drain(semg, sub)
            fire_scatters(sub)

            @pl.when(g < NCHUNK - 1)
            def _prep():
                @pl.when(g >= 1)
                def _free():
                    drain(sems[1 - sub], 1 - sub)
                stage_ids(g + 1, 1 - sub)
                fire_gathers(1 - sub)

    drain(sems[0], 0)
    drain(sems[1], 1)

    for p in range(G // 128):
        pltpu.sync_copy(acc_sp.at[pl.ds(slot0 + p * 128, 128)], obuf)

        @pl.loop(0, 128)
        def _scale(b):
            inv = plsc.load_gather(inv_v, [jnp.broadcast_to(p * 128 + b, (16,))])
            for h in range(D // 16):
                obuf[b, pl.ds(h * 16, 16)] = obuf[b, pl.ds(h * 16, 16)] * inv

        pltpu.sync_copy(obuf, out_hbm.at[pl.ds(base_b + p * 128, 128)])


NFULL = 7812
TAIL = VOCAB - NFULL * 128


def _tr_body(embt_hbm, tail_hbm, zero_hbm, out_hbm,
             st_a, st_b, st2_a, st2_b, tailv, zv, semi, semo):
    c = lax.axis_index("c")
    s = lax.axis_index("s")
    wid = c * NS + s
    base = NFULL // 32
    extra = NFULL - 32 * base
    nb = jnp.where(wid < extra, base + 1, base)
    start = wid * base + jnp.minimum(wid, extra)

    st = (st_a, st_b)
    st2 = (st2_a, st2_b)
    iota32 = lax.iota(jnp.int32, 16) * D

    def load_block(bi, sub):
        for dt in range(D // 8):
            pltpu.async_copy(
                embt_hbm.at[pl.ds(dt * 8, 8), pl.ds(bi * 128, 128)],
                st[sub].at[pl.ds(dt * 8, 8)], semi)

    def drain_in(sub):
        for dt in range(D // 8):
            pltpu.make_async_copy(
                embt_hbm.at[pl.ds(dt * 8, 8), pl.ds(0, 128)],
                st[sub].at[pl.ds(dt * 8, 8)], semi).wait()

    def drain_out(sub):
        pltpu.make_async_copy(out_hbm.at[pl.ds(0, 128 * D)],
                              st2[sub], semo).wait()

    pltpu.sync_copy(zero_hbm, zv)
    iota_z = lax.iota(jnp.int32, 16) + zv[pl.ds(0, 16)]

    def transpose_block(sub):
        for d0 in range(0, D, 16):
            drow = iota_z + d0

            @pl.loop(0, 16, unroll=8)
            def _j(j):
                perm = (iota_z + j) & 15
                st_base = perm * D + drow
                for cg in range(8):
                    c0 = cg * 16
                    x = plsc.load_gather(st[sub], [drow, perm + c0])
                    plsc.store_scatter(st2[sub], [st_base + c0 * D], x)

    def store_block(bi, sub):
        pltpu.async_copy(st2[sub],
                         out_hbm.at[pl.ds(bi * (128 * D), 128 * D)], semo)

    load_block(start, 0)

    @pl.loop(0, base + 1, step=2)
    def _blk(t):
        for sub in range(2):
            g = t + sub

            @pl.when(g < nb)
            def _():
                drain_in(sub)
                @pl.when(g + 1 < nb)
                def _():
                    load_block(start + g + 1, 1 - sub)
                @pl.when(g >= 2)
                def _():
                    drain_out(sub)
                transpose_block(sub)
                store_block(start + g, sub)

    drain_out(0)
    drain_out(1)

    @pl.when(wid == 31)
    def _():
        pltpu.sync_copy(tail_hbm, tailv)
        pltpu.sync_copy(tailv, out_hbm.at[pl.ds(NFULL * 128 * D, TAIL * D)])


_transpose = pl.kernel(
    _tr_body,
    out_type=jax.ShapeDtypeStruct((VOCAB * D,), jnp.float32),
    mesh=plsc.VectorSubcoreMesh(core_axis_name="c", subcore_axis_name="s"),
    compiler_params=pltpu.CompilerParams(
        needs_layout_passes=False, use_tc_tiling_on_sc=True),
    scratch_types=[
        pltpu.VMEM((D, 128), jnp.float32),
        pltpu.VMEM((D, 128), jnp.float32),
        pltpu.VMEM((128 * D,), jnp.float32),
        pltpu.VMEM((128 * D,), jnp.float32),
        pltpu.VMEM((TAIL * D,), jnp.float32),
        pltpu.VMEM((16,), jnp.int32),
        pltpu.SemaphoreType.DMA,
        pltpu.SemaphoreType.DMA,
    ],
)


_bag = pl.kernel(
    _body,
    out_type=jax.ShapeDtypeStruct((B, D), jnp.float32),
    mesh=plsc.VectorSubcoreMesh(core_axis_name="c", subcore_axis_name="s"),
    compiler_params=pltpu.CompilerParams(
        needs_layout_passes=False, use_tc_tiling_on_sc=False),
    scratch_types=[
        pltpu.VMEM_SHARED((NS * G + NS, D), jnp.float32),
        pltpu.VMEM((2, CHUNK // 128, 128), jnp.int32),
        pltpu.VMEM((2, CHUNK // 128, 128), jnp.int32),
        pltpu.VMEM((2, CHUNK, D), jnp.float32),
        pltpu.VMEM((G,), jnp.int32),
        pltpu.VMEM((G,), jnp.float32),
        pltpu.VMEM((128, D), jnp.float32),
        pltpu.SemaphoreType.DMA,
        pltpu.SemaphoreType.DMA,
        pltpu.SemaphoreType.DMA,
    ],
)


@jax.jit
def kernel(ngram_ids, ngram_lengths, embedding):
    ids2d = ngram_ids.reshape(B * L // 128, 128)
    emb_t = embedding.T
    tail_flat = embedding[NFULL * 128:, :].reshape(TAIL * D)
    emb_lin = _transpose(emb_t, tail_flat, jnp.zeros((16,), jnp.int32))
    return _bag(ids2d, ngram_lengths, emb_lin.reshape(VOCAB, D))
